# baseline (speedup 1.0000x reference)
import jax
import jax.numpy as jnp
from jax.experimental import pallas as pl
from jax.experimental.pallas import tpu as pltpu

_N = 10000
_BN = 1000
_DH = 128


def _fused_kernel(x_ref, wcat_ref, bcat_ref, wco_ref,
                  w1_ref, b1_ref, w2_ref, b2_ref, w3_ref, b3_ref, out_ref):
    xb = x_ref[...]
    g = jnp.dot(xb, wcat_ref[...], preferred_element_type=jnp.float32)
    g = g + bcat_ref[...]
    i = jax.nn.sigmoid(g[:, 0:_DH])
    t = jnp.tanh(g[:, _DH:2 * _DH])
    c = i * t
    o = jax.nn.sigmoid(g[:, 2 * _DH:3 * _DH] + wco_ref[...] * c)
    h = jax.nn.relu(o * jnp.tanh(c))
    h1 = jax.nn.relu(
        jnp.dot(h, w1_ref[...], preferred_element_type=jnp.float32) + b1_ref[...])
    h2 = jax.nn.relu(
        jnp.dot(h1, w2_ref[...], preferred_element_type=jnp.float32) + b2_ref[...])
    out_ref[...] = (
        jnp.dot(h2, w3_ref[...], preferred_element_type=jnp.float32) + b3_ref[...])


def kernel(x, edge_index, edge_weight, W_i, W_f, W_c, W_o, conv_i_w, conv_i_b,
           conv_f_w, conv_f_b, conv_c_w, conv_c_b, conv_o_w, conv_o_b,
           w_ci, w_cf, w_co, b_i, b_f, b_c, b_o,
           mlp1_w, mlp1_b, mlp2_w, mlp2_b, mlp3_w, mlp3_b):
    x = x.astype(jnp.float32)
    wcat = jnp.concatenate([W_i, W_c, W_o], axis=1)
    bconv = jnp.concatenate([conv_i_b, conv_c_b, conv_o_b])[None]
    bgate = jnp.concatenate([b_i, b_c, b_o], axis=1)
    bcat = bconv + bgate

    grid = _N // _BN
    full = lambda i: (0, 0)
    out = pl.pallas_call(
        _fused_kernel,
        grid=(grid,),
        in_specs=[
            pl.BlockSpec((_BN, _DH), lambda i: (i, 0)),
            pl.BlockSpec((_DH, 3 * _DH), full),
            pl.BlockSpec((1, 3 * _DH), full),
            pl.BlockSpec((1, _DH), full),
            pl.BlockSpec((_DH, _DH // 2), full),
            pl.BlockSpec((1, _DH // 2), full),
            pl.BlockSpec((_DH // 2, _DH // 4), full),
            pl.BlockSpec((1, _DH // 4), full),
            pl.BlockSpec((_DH // 4, 1), full),
            pl.BlockSpec((1, 1), full),
        ],
        out_specs=pl.BlockSpec((_BN, 1), lambda i: (i, 0)),
        out_shape=jax.ShapeDtypeStruct((_N, 1), jnp.float32),
        compiler_params=pltpu.CompilerParams(
            dimension_semantics=("arbitrary",),
        ),
    )(x, wcat, bcat, w_co, mlp1_w, mlp1_b[None], mlp2_w, mlp2_b[None],
      mlp3_w, mlp3_b[None])
    return jnp.squeeze(out, axis=-1)


# speedup vs baseline: 1.1456x; 1.1456x over previous
import jax
import jax.numpy as jnp
from jax.experimental import pallas as pl
from jax.experimental.pallas import tpu as pltpu

_N = 10000
_BN = 2000
_DH = 128


def _fused_kernel(x_ref, wcat_ref, bcat_ref, wco_ref,
                  w1_ref, b1_ref, w2_ref, b2_ref, w3_ref, b3_ref, out_ref):
    xb = x_ref[...]
    g = jnp.dot(xb, wcat_ref[...], preferred_element_type=jnp.float32)
    g = g + bcat_ref[...]
    i = jax.nn.sigmoid(g[:, 0:_DH])
    t = jnp.tanh(g[:, _DH:2 * _DH])
    c = i * t
    o = jax.nn.sigmoid(g[:, 2 * _DH:3 * _DH] + wco_ref[...] * c)
    h = jax.nn.relu(o * jnp.tanh(c))
    h1 = jax.nn.relu(
        jnp.dot(h, w1_ref[...], preferred_element_type=jnp.float32) + b1_ref[...])
    h2 = jax.nn.relu(
        jnp.dot(h1, w2_ref[...], preferred_element_type=jnp.float32) + b2_ref[...])
    out_ref[...] = (
        jnp.dot(h2, w3_ref[...], preferred_element_type=jnp.float32) + b3_ref[...])


def kernel(x, edge_index, edge_weight, W_i, W_f, W_c, W_o, conv_i_w, conv_i_b,
           conv_f_w, conv_f_b, conv_c_w, conv_c_b, conv_o_w, conv_o_b,
           w_ci, w_cf, w_co, b_i, b_f, b_c, b_o,
           mlp1_w, mlp1_b, mlp2_w, mlp2_b, mlp3_w, mlp3_b):
    x = x.astype(jnp.float32)
    wcat = jnp.concatenate([W_i, W_c, W_o], axis=1)
    bconv = jnp.concatenate([conv_i_b, conv_c_b, conv_o_b])[None]
    bgate = jnp.concatenate([b_i, b_c, b_o], axis=1)
    bcat = bconv + bgate

    grid = _N // _BN
    full = lambda i: (0, 0)
    out = pl.pallas_call(
        _fused_kernel,
        grid=(grid,),
        in_specs=[
            pl.BlockSpec((_BN, _DH), lambda i: (i, 0)),
            pl.BlockSpec((_DH, 3 * _DH), full),
            pl.BlockSpec((1, 3 * _DH), full),
            pl.BlockSpec((1, _DH), full),
            pl.BlockSpec((_DH, _DH // 2), full),
            pl.BlockSpec((1, _DH // 2), full),
            pl.BlockSpec((_DH // 2, _DH // 4), full),
            pl.BlockSpec((1, _DH // 4), full),
            pl.BlockSpec((_DH // 4, 1), full),
            pl.BlockSpec((1, 1), full),
        ],
        out_specs=pl.BlockSpec((_BN, 1), lambda i: (i, 0)),
        out_shape=jax.ShapeDtypeStruct((_N, 1), jnp.float32),
        compiler_params=pltpu.CompilerParams(
            dimension_semantics=("arbitrary",),
        ),
    )(x, wcat, bcat, w_co, mlp1_w, mlp1_b[None], mlp2_w, mlp2_b[None],
      mlp3_w, mlp3_b[None])
    return jnp.squeeze(out, axis=-1)


# speedup vs baseline: 1.1504x; 1.0042x over previous
import jax
import jax.numpy as jnp
from jax.experimental import pallas as pl
from jax.experimental.pallas import tpu as pltpu

_N = 10000
_BN = 2000
_DH = 128


def _fused_kernel(x_ref, wcat_ref, bcat_ref, wco_ref,
                  w1_ref, b1_ref, w2_ref, b2_ref, w3_ref, b3_ref, out_ref):
    xb = x_ref[...]
    g = jnp.dot(xb.astype(jnp.bfloat16), wcat_ref[...].astype(jnp.bfloat16),
                preferred_element_type=jnp.float32)
    g = g + bcat_ref[...]
    i = jax.nn.sigmoid(g[:, 0:_DH])
    t = jnp.tanh(g[:, _DH:2 * _DH])
    c = i * t
    o = jax.nn.sigmoid(g[:, 2 * _DH:3 * _DH] + wco_ref[...] * c)
    h = jax.nn.relu(o * jnp.tanh(c))
    h1 = jax.nn.relu(
        jnp.dot(h, w1_ref[...], preferred_element_type=jnp.float32) + b1_ref[...])
    h2 = jax.nn.relu(
        jnp.dot(h1, w2_ref[...], preferred_element_type=jnp.float32) + b2_ref[...])
    out_ref[...] = (
        jnp.dot(h2, w3_ref[...], preferred_element_type=jnp.float32) + b3_ref[...])


def kernel(x, edge_index, edge_weight, W_i, W_f, W_c, W_o, conv_i_w, conv_i_b,
           conv_f_w, conv_f_b, conv_c_w, conv_c_b, conv_o_w, conv_o_b,
           w_ci, w_cf, w_co, b_i, b_f, b_c, b_o,
           mlp1_w, mlp1_b, mlp2_w, mlp2_b, mlp3_w, mlp3_b):
    x = x.astype(jnp.float32)
    wcat = jnp.concatenate([W_i, W_c, W_o], axis=1)
    bconv = jnp.concatenate([conv_i_b, conv_c_b, conv_o_b])[None]
    bgate = jnp.concatenate([b_i, b_c, b_o], axis=1)
    bcat = bconv + bgate

    grid = _N // _BN
    full = lambda i: (0, 0)
    out = pl.pallas_call(
        _fused_kernel,
        grid=(grid,),
        in_specs=[
            pl.BlockSpec((_BN, _DH), lambda i: (i, 0)),
            pl.BlockSpec((_DH, 3 * _DH), full),
            pl.BlockSpec((1, 3 * _DH), full),
            pl.BlockSpec((1, _DH), full),
            pl.BlockSpec((_DH, _DH // 2), full),
            pl.BlockSpec((1, _DH // 2), full),
            pl.BlockSpec((_DH // 2, _DH // 4), full),
            pl.BlockSpec((1, _DH // 4), full),
            pl.BlockSpec((_DH // 4, 1), full),
            pl.BlockSpec((1, 1), full),
        ],
        out_specs=pl.BlockSpec((_BN, 1), lambda i: (i, 0)),
        out_shape=jax.ShapeDtypeStruct((_N, 1), jnp.float32),
        compiler_params=pltpu.CompilerParams(
            dimension_semantics=("arbitrary",),
        ),
    )(x, wcat, bcat, w_co, mlp1_w, mlp1_b[None], mlp2_w, mlp2_b[None],
      mlp3_w, mlp3_b[None])
    return jnp.squeeze(out, axis=-1)
